# baseline (speedup 1.0000x reference)
import functools

import jax
import jax.numpy as jnp
from jax import lax
from jax.experimental import pallas as pl
from jax.experimental.pallas import tpu as pltpu
from jax.experimental.pallas import tpu_sc as plsc

_B, _L, _D = 16384, 50, 32
_AVG = 3.5
_NC, _NS = 2, 16
_NW = _NC * _NS
_PER_W = _B // _NW
_CHUNK = 16
_NCHUNK = _PER_W // _CHUNK
_ROWS = _CHUNK * _L
_INV_SQRT_L = float(1.0 / (_L ** 0.5))
_ACH = 4
_NACH = _PER_W // _ACH


def _full(v):
    return jnp.full((16,), v, jnp.int32)


def _ui_body(uid_hbm, iid_hbm, utT_hbm, itT_hbm, bias_hbm,
             part1_hbm, irows_hbm,
             uid_v, iid_v, ubuf_v, ibuf_v, bias_v, p1_v, usT_v, irT_v,
             sem_u0, sem_u1, sem_i0, sem_i1):
    wid = lax.axis_index("s") * _NC + lax.axis_index("c")
    base = wid * _PER_W
    usems = (sem_u0, sem_u1)
    isems = (sem_i0, sem_i1)
    lane = lax.iota(jnp.int32, 16)

    pltpu.sync_copy(bias_hbm, bias_v)
    bias_vec = bias_v[...]
    pltpu.sync_copy(uid_hbm.at[pl.ds(base, _PER_W)], uid_v)
    pltpu.sync_copy(iid_hbm.at[pl.ds(base, _PER_W)], iid_v)

    def start(c, b):
        for e in range(_ACH):
            us = plsc.load_gather(uid_v, [_full(c * _ACH + e)])[0]
            s2 = plsc.load_gather(iid_v, [_full(c * _ACH + e)])[0]
            uct = pl.multiple_of(
                lax.shift_left(lax.shift_right_logical(us, 7), 7), 128)
            ict = pl.multiple_of(
                lax.shift_left(lax.shift_right_logical(s2, 7), 7), 128)
            for k in range(4):
                pltpu.async_copy(
                    utT_hbm.at[pl.ds(8 * k, 8), pl.ds(uct, 128)],
                    ubuf_v.at[b, e, k], usems[b])
                pltpu.async_copy(
                    itT_hbm.at[pl.ds(8 * k, 8), pl.ds(ict, 128)],
                    ibuf_v.at[b, e, k], isems[b])

    start(0, 0)
    start(1, 1)

    def outer(o, carry):
        for b in range(2):
            c = 2 * o + b
            for e in range(_ACH):
                for k in range(4):
                    pltpu.make_async_copy(
                        utT_hbm.at[pl.ds(0, 8), pl.ds(0, 128)],
                        ubuf_v.at[b, e, k], usems[b]).wait()
                    pltpu.make_async_copy(
                        itT_hbm.at[pl.ds(0, 8), pl.ds(0, 128)],
                        ibuf_v.at[b, e, k], isems[b]).wait()

            for e in range(_ACH):
                g = c * _ACH + e
                ucol = jnp.bitwise_and(
                    plsc.load_gather(uid_v, [_full(g)]), 127)
                icol = jnp.bitwise_and(
                    plsc.load_gather(iid_v, [_full(g)]), 127)
                gv = _full(g)
                k_lo = lax.shift_right_logical(lane, 3)
                d8 = jnp.bitwise_and(lane, 7)
                k_hi = k_lo + 2
                u_lo = plsc.load_gather(ubuf_v.at[b, e], [k_lo, d8, ucol])
                u_hi = plsc.load_gather(ubuf_v.at[b, e], [k_hi, d8, ucol])
                i_lo = plsc.load_gather(ibuf_v.at[b, e], [k_lo, d8, icol])
                i_hi = plsc.load_gather(ibuf_v.at[b, e], [k_hi, d8, icol])
                plsc.store_scatter(usT_v, [lane * _PER_W + gv], u_lo)
                plsc.store_scatter(usT_v, [(lane + 16) * _PER_W + gv], u_hi)
                plsc.store_scatter(irT_v, [lane * _PER_W + gv], i_lo)
                plsc.store_scatter(irT_v, [(lane + 16) * _PER_W + gv], i_hi)

            @pl.when(c + 2 < _NACH)
            def _(b=b, c=c):
                start(c + 2, b)
        return carry

    lax.fori_loop(0, _NACH // 2, outer, 0)

    def dot_body(c, carry):
        acc_a = bias_vec
        acc_b = jnp.zeros((16,), jnp.float32)
        for d in range(0, _D, 2):
            uv0 = usT_v[pl.ds(d * _PER_W + c * _CHUNK, _CHUNK)]
            iv0 = irT_v[pl.ds(d * _PER_W + c * _CHUNK, _CHUNK)]
            uv1 = usT_v[pl.ds((d + 1) * _PER_W + c * _CHUNK, _CHUNK)]
            iv1 = irT_v[pl.ds((d + 1) * _PER_W + c * _CHUNK, _CHUNK)]
            acc_a = acc_a + uv0 * iv0
            acc_b = acc_b + uv1 * iv1
        p1_v[pl.ds(c * _CHUNK, _CHUNK)] = acc_a + acc_b
        return carry

    lax.fori_loop(0, _NCHUNK, dot_body, 0)
    pltpu.sync_copy(p1_v, part1_hbm.at[pl.ds(base, _PER_W)])
    pltpu.sync_copy(irT_v, irows_hbm.at[pl.ds(base * _D, _PER_W * _D)])


def _hist_body(hid_hbm, ht_hbm, irows_hbm,
               part2_hbm,
               hid_v, hrow_v, ir_v, p2_v, hpool_v,
               sem_h0, sem_h1):
    wid = lax.axis_index("s") * _NC + lax.axis_index("c")
    base = wid * _PER_W
    hbase = base * _L
    sems = (sem_h0, sem_h1)
    lane = lax.iota(jnp.int32, 16)

    pltpu.sync_copy(irows_hbm.at[pl.ds(base * _D, _PER_W * _D)], ir_v)

    def start(c, b):
        pltpu.sync_copy(hid_hbm.at[pl.ds(hbase + c * _ROWS, _ROWS)],
                        hid_v.at[b])
        pltpu.async_copy(ht_hbm.at[hid_v.at[b]], hrow_v.at[b], sems[b])

    start(0, 0)
    start(1, 1)

    def outer(o, carry):
        for b in range(2):
            c = 2 * o + b
            pltpu.make_async_copy(ht_hbm.at[hid_v.at[b]], hrow_v.at[b],
                                  sems[b]).wait()

            def ex_body(e, carry2, b=b):
                r0 = e * _L
                a0 = jnp.zeros((16,), jnp.float32)
                a1 = jnp.zeros((16,), jnp.float32)
                a2 = jnp.zeros((16,), jnp.float32)
                a3 = jnp.zeros((16,), jnp.float32)
                for l in range(0, _L, 2):
                    a0 = a0 + hrow_v[b, r0 + l, pl.ds(0, 16)]
                    a1 = a1 + hrow_v[b, r0 + l, pl.ds(16, 16)]
                    a2 = a2 + hrow_v[b, r0 + l + 1, pl.ds(0, 16)]
                    a3 = a3 + hrow_v[b, r0 + l + 1, pl.ds(16, 16)]
                hpool_v[e, pl.ds(0, 16)] = a0 + a2
                hpool_v[e, pl.ds(16, 16)] = a1 + a3
                return carry2

            lax.fori_loop(0, _CHUNK, ex_body, 0)

            acc_a = jnp.zeros((16,), jnp.float32)
            acc_b = jnp.zeros((16,), jnp.float32)
            for d in range(0, _D, 2):
                hv0 = plsc.load_gather(hpool_v, [lane, _full(d)])
                hv1 = plsc.load_gather(hpool_v, [lane, _full(d + 1)])
                iv0 = ir_v[pl.ds(d * _PER_W + c * _CHUNK, _CHUNK)]
                iv1 = ir_v[pl.ds((d + 1) * _PER_W + c * _CHUNK, _CHUNK)]
                acc_a = acc_a + hv0 * iv0
                acc_b = acc_b + hv1 * iv1
            p2_v[pl.ds(c * _CHUNK, _CHUNK)] = (acc_a + acc_b) * _INV_SQRT_L

            @pl.when(c + 2 < _NCHUNK)
            def _(b=b, c=c):
                start(c + 2, b)
        return carry

    lax.fori_loop(0, _NCHUNK // 2, outer, 0)
    pltpu.sync_copy(p2_v, part2_hbm.at[pl.ds(base, _PER_W)])


@jax.jit
def _svdpp(user_ids, item_ids, hist_flat, user_table, item_table,
           hist_table, bias_vec):
    mesh = plsc.VectorSubcoreMesh(core_axis_name="c", subcore_axis_name="s")

    ui = pl.kernel(
        _ui_body,
        out_type=[
            jax.ShapeDtypeStruct((_B,), jnp.float32),
            jax.ShapeDtypeStruct((_B * _D,), jnp.float32),
        ],
        mesh=mesh,
        compiler_params=pltpu.CompilerParams(
            needs_layout_passes=False, use_tc_tiling_on_sc=True),
        scratch_types=[
            pltpu.VMEM((_PER_W,), jnp.int32),
            pltpu.VMEM((_PER_W,), jnp.int32),
            pltpu.VMEM((2, _ACH, 4, 8, 128), jnp.float32),
            pltpu.VMEM((2, _ACH, 4, 8, 128), jnp.float32),
            pltpu.VMEM((16,), jnp.float32),
            pltpu.VMEM((_PER_W,), jnp.float32),
            pltpu.VMEM((_PER_W * _D,), jnp.float32),
            pltpu.VMEM((_PER_W * _D,), jnp.float32),
            pltpu.SemaphoreType.DMA,
            pltpu.SemaphoreType.DMA,
            pltpu.SemaphoreType.DMA,
            pltpu.SemaphoreType.DMA,
        ],
    )
    part1, irows = ui(user_ids, item_ids, user_table.T, item_table.T,
                      bias_vec)

    hist = pl.kernel(
        _hist_body,
        out_type=jax.ShapeDtypeStruct((_B,), jnp.float32),
        mesh=mesh,
        compiler_params=pltpu.CompilerParams(
            needs_layout_passes=False, use_tc_tiling_on_sc=False),
        scratch_types=[
            pltpu.VMEM((2, _ROWS), jnp.int32),
            pltpu.VMEM((2, _ROWS, _D), jnp.float32),
            pltpu.VMEM((_PER_W * _D,), jnp.float32),
            pltpu.VMEM((_PER_W,), jnp.float32),
            pltpu.VMEM((_CHUNK, _D), jnp.float32),
            pltpu.SemaphoreType.DMA,
            pltpu.SemaphoreType.DMA,
        ],
    )
    part2 = hist(hist_flat, hist_table, irows)
    return part1 + part2


def kernel(user_ids, item_ids, hist_ids, user_table, item_table, hist_table,
           user_bias, item_bias):
    bias = _AVG + user_bias[0] + item_bias[0]
    bias_vec = jnp.full((16,), bias, jnp.float32)
    hist_flat = hist_ids.reshape(-1)
    return _svdpp(user_ids, item_ids, hist_flat, user_table, item_table,
                  hist_table, bias_vec)


# speedup vs baseline: 1.0036x; 1.0036x over previous
import functools

import jax
import jax.numpy as jnp
from jax import lax
from jax.experimental import pallas as pl
from jax.experimental.pallas import tpu as pltpu
from jax.experimental.pallas import tpu_sc as plsc

_B, _L, _D = 16384, 50, 32
_AVG = 3.5
_NC, _NS = 2, 16
_NW = _NC * _NS
_PER_W = _B // _NW
_CHUNK = 16
_NCHUNK = _PER_W // _CHUNK
_ROWS = _CHUNK * _L
_INV_SQRT_L = float(1.0 / (_L ** 0.5))
_ACH = 4
_NACH = _PER_W // _ACH


def _full(v):
    return jnp.full((16,), v, jnp.int32)


def _ui_body(uid_hbm, iid_hbm, utT_hbm, itT_hbm, bias_hbm,
             part1_hbm, irows_hbm,
             uid_v, iid_v, ubuf_v, ibuf_v, bias_v, p1_v, usT_v, irT_v,
             sem_u0, sem_u1, sem_i0, sem_i1):
    wid = lax.axis_index("s") * _NC + lax.axis_index("c")
    base = wid * _PER_W
    usems = (sem_u0, sem_u1)
    isems = (sem_i0, sem_i1)
    lane = lax.iota(jnp.int32, 16)

    pltpu.sync_copy(bias_hbm, bias_v)
    bias_vec = bias_v[...]
    pltpu.sync_copy(uid_hbm.at[pl.ds(base, _PER_W)], uid_v)
    pltpu.sync_copy(iid_hbm.at[pl.ds(base, _PER_W)], iid_v)

    def start(c, b):
        for e in range(_ACH):
            us = plsc.load_gather(uid_v, [_full(c * _ACH + e)])[0]
            s2 = plsc.load_gather(iid_v, [_full(c * _ACH + e)])[0]
            uct = pl.multiple_of(
                lax.shift_left(lax.shift_right_logical(us, 7), 7), 128)
            ict = pl.multiple_of(
                lax.shift_left(lax.shift_right_logical(s2, 7), 7), 128)
            for k in range(4):
                pltpu.async_copy(
                    utT_hbm.at[pl.ds(8 * k, 8), pl.ds(uct, 128)],
                    ubuf_v.at[b, e, k], usems[b])
                pltpu.async_copy(
                    itT_hbm.at[pl.ds(8 * k, 8), pl.ds(ict, 128)],
                    ibuf_v.at[b, e, k], isems[b])

    start(0, 0)
    start(1, 1)

    def outer(o, carry):
        for b in range(2):
            c = 2 * o + b
            for e in range(_ACH):
                for k in range(4):
                    pltpu.make_async_copy(
                        utT_hbm.at[pl.ds(0, 8), pl.ds(0, 128)],
                        ubuf_v.at[b, e, k], usems[b]).wait()
                    pltpu.make_async_copy(
                        itT_hbm.at[pl.ds(0, 8), pl.ds(0, 128)],
                        ibuf_v.at[b, e, k], isems[b]).wait()

            for e in range(_ACH):
                g = c * _ACH + e
                ucol = jnp.bitwise_and(
                    plsc.load_gather(uid_v, [_full(g)]), 127)
                icol = jnp.bitwise_and(
                    plsc.load_gather(iid_v, [_full(g)]), 127)
                gv = _full(g)
                k_lo = lax.shift_right_logical(lane, 3)
                d8 = jnp.bitwise_and(lane, 7)
                k_hi = k_lo + 2
                u_lo = plsc.load_gather(ubuf_v.at[b, e], [k_lo, d8, ucol])
                u_hi = plsc.load_gather(ubuf_v.at[b, e], [k_hi, d8, ucol])
                i_lo = plsc.load_gather(ibuf_v.at[b, e], [k_lo, d8, icol])
                i_hi = plsc.load_gather(ibuf_v.at[b, e], [k_hi, d8, icol])
                plsc.store_scatter(usT_v, [lane * _PER_W + gv], u_lo)
                plsc.store_scatter(usT_v, [(lane + 16) * _PER_W + gv], u_hi)
                plsc.store_scatter(irT_v, [lane * _PER_W + gv], i_lo)
                plsc.store_scatter(irT_v, [(lane + 16) * _PER_W + gv], i_hi)

            @pl.when(c + 2 < _NACH)
            def _(b=b, c=c):
                start(c + 2, b)
        return carry

    lax.fori_loop(0, _NACH // 2, outer, 0)

    def dot_body(c, carry):
        acc_a = bias_vec
        acc_b = jnp.zeros((16,), jnp.float32)
        for d in range(0, _D, 2):
            uv0 = usT_v[pl.ds(d * _PER_W + c * _CHUNK, _CHUNK)]
            iv0 = irT_v[pl.ds(d * _PER_W + c * _CHUNK, _CHUNK)]
            uv1 = usT_v[pl.ds((d + 1) * _PER_W + c * _CHUNK, _CHUNK)]
            iv1 = irT_v[pl.ds((d + 1) * _PER_W + c * _CHUNK, _CHUNK)]
            acc_a = acc_a + uv0 * iv0
            acc_b = acc_b + uv1 * iv1
        p1_v[pl.ds(c * _CHUNK, _CHUNK)] = acc_a + acc_b
        return carry

    lax.fori_loop(0, _NCHUNK, dot_body, 0)
    pltpu.sync_copy(p1_v, part1_hbm.at[pl.ds(base, _PER_W)])
    pltpu.sync_copy(irT_v, irows_hbm.at[pl.ds(base * _D, _PER_W * _D)])


def _hist_body(hid_hbm, ht_hbm, irows_hbm,
               part2_hbm,
               hid_v, hrow_v, ir_v, p2_v, hpool_v,
               sem_h0, sem_h1):
    wid = lax.axis_index("s") * _NC + lax.axis_index("c")
    base = wid * _PER_W
    hbase = base * _L
    sems = (sem_h0, sem_h1)
    lane = lax.iota(jnp.int32, 16)

    pltpu.sync_copy(irows_hbm.at[pl.ds(base * _D, _PER_W * _D)], ir_v)

    def start(c, b):
        pltpu.sync_copy(hid_hbm.at[pl.ds(hbase + c * _ROWS, _ROWS)],
                        hid_v.at[b])
        pltpu.async_copy(ht_hbm.at[hid_v.at[b]], hrow_v.at[b], sems[b])

    start(0, 0)
    start(1, 1)

    def outer(o, carry):
        for b in range(2):
            c = 2 * o + b
            pltpu.make_async_copy(ht_hbm.at[hid_v.at[b]], hrow_v.at[b],
                                  sems[b]).wait()

            def ex_body(e, carry2, b=b):
                r0 = e * _L
                a0 = jnp.zeros((16,), jnp.float32)
                a1 = jnp.zeros((16,), jnp.float32)
                a2 = jnp.zeros((16,), jnp.float32)
                a3 = jnp.zeros((16,), jnp.float32)
                for l in range(0, _L, 2):
                    a0 = a0 + hrow_v[b, r0 + l, pl.ds(0, 16)]
                    a1 = a1 + hrow_v[b, r0 + l, pl.ds(16, 16)]
                    a2 = a2 + hrow_v[b, r0 + l + 1, pl.ds(0, 16)]
                    a3 = a3 + hrow_v[b, r0 + l + 1, pl.ds(16, 16)]
                hpool_v[e, pl.ds(0, 16)] = a0 + a2
                hpool_v[e, pl.ds(16, 16)] = a1 + a3
                return carry2

            lax.fori_loop(0, _CHUNK, ex_body, 0)

            acc_a = jnp.zeros((16,), jnp.float32)
            acc_b = jnp.zeros((16,), jnp.float32)
            for d in range(0, _D, 2):
                hv0 = plsc.load_gather(hpool_v, [lane, _full(d)])
                hv1 = plsc.load_gather(hpool_v, [lane, _full(d + 1)])
                iv0 = ir_v[pl.ds(d * _PER_W + c * _CHUNK, _CHUNK)]
                iv1 = ir_v[pl.ds((d + 1) * _PER_W + c * _CHUNK, _CHUNK)]
                acc_a = acc_a + hv0 * iv0
                acc_b = acc_b + hv1 * iv1
            p2_v[pl.ds(c * _CHUNK, _CHUNK)] = (acc_a + acc_b) * _INV_SQRT_L

            @pl.when(c + 2 < _NCHUNK)
            def _(b=b, c=c):
                start(c + 2, b)
        return carry

    lax.fori_loop(0, _NCHUNK // 2, outer, 0)
    pltpu.sync_copy(p2_v, part2_hbm.at[pl.ds(base, _PER_W)])


@jax.jit
def _svdpp(user_ids, item_ids, hist_flat, user_table, item_table,
           hist_table, bias_vec):
    mesh = plsc.VectorSubcoreMesh(core_axis_name="c", subcore_axis_name="s")

    ui = pl.kernel(
        _ui_body,
        out_type=[
            jax.ShapeDtypeStruct((_B,), jnp.float32),
            jax.ShapeDtypeStruct((_B * _D,), jnp.float32),
        ],
        mesh=mesh,
        compiler_params=pltpu.CompilerParams(
            needs_layout_passes=False, use_tc_tiling_on_sc=True),
        scratch_types=[
            pltpu.VMEM((_PER_W,), jnp.int32),
            pltpu.VMEM((_PER_W,), jnp.int32),
            pltpu.VMEM((2, _ACH, 4, 8, 128), jnp.float32),
            pltpu.VMEM((2, _ACH, 4, 8, 128), jnp.float32),
            pltpu.VMEM((16,), jnp.float32),
            pltpu.VMEM((_PER_W,), jnp.float32),
            pltpu.VMEM((_PER_W * _D,), jnp.float32),
            pltpu.VMEM((_PER_W * _D,), jnp.float32),
            pltpu.SemaphoreType.DMA,
            pltpu.SemaphoreType.DMA,
            pltpu.SemaphoreType.DMA,
            pltpu.SemaphoreType.DMA,
        ],
    )
    part1, irows = ui(user_ids, item_ids, user_table.T, item_table.T,
                      bias_vec)

    hist = pl.kernel(
        _hist_body,
        out_type=jax.ShapeDtypeStruct((_B,), jnp.float32),
        mesh=mesh,
        compiler_params=pltpu.CompilerParams(
            needs_layout_passes=False, use_tc_tiling_on_sc=False),
        scratch_types=[
            pltpu.VMEM((2, _ROWS), jnp.int32),
            pltpu.VMEM((2, _ROWS, _D), jnp.float32),
            pltpu.VMEM((_PER_W * _D,), jnp.float32),
            pltpu.VMEM((_PER_W,), jnp.float32),
            pltpu.VMEM((_CHUNK, _D), jnp.float32),
            pltpu.SemaphoreType.DMA,
            pltpu.SemaphoreType.DMA,
        ],
    )
    hist_lin = lax.optimization_barrier(hist_table.reshape(-1))
    part2 = hist(hist_flat, hist_lin.reshape(1000000, _D), irows)
    return part1 + part2


def kernel(user_ids, item_ids, hist_ids, user_table, item_table, hist_table,
           user_bias, item_bias):
    bias = _AVG + user_bias[0] + item_bias[0]
    bias_vec = jnp.full((16,), bias, jnp.float32)
    hist_flat = hist_ids.reshape(-1)
    return _svdpp(user_ids, item_ids, hist_flat, user_table, item_table,
                  hist_table, bias_vec)
